# baseline (speedup 1.0000x reference)
import functools

import jax
import jax.numpy as jnp
from jax import lax
from jax.experimental import pallas as pl
from jax.experimental.pallas import tpu as pltpu
from jax.experimental.pallas import tpu_sc as plsc

N_ROWS = 1000000
CODE_LEN = 64
BATCH = 16384

_info = plsc.get_sparse_core_info()
_NC, _NS = _info.num_cores, _info.num_subcores
_NW = _NC * _NS
_B_PER_W = BATCH // _NW
_HALF = _B_PER_W // 2
_SP_PER_SC = _HALF * _NS

_mesh = plsc.VectorSubcoreMesh(core_axis_name="c", subcore_axis_name="s")


@functools.partial(
    pl.kernel,
    mesh=_mesh,
    out_type=jax.ShapeDtypeStruct((BATCH, CODE_LEN), jnp.float32),
    scratch_types=[
        pltpu.VMEM((_B_PER_W,), jnp.int32),
        pltpu.VMEM((_HALF, CODE_LEN), jnp.float32),
        pltpu.VMEM_SHARED((_SP_PER_SC, CODE_LEN), jnp.float32),
        pltpu.SemaphoreType.DMA,
        pltpu.SemaphoreType.DMA,
        pltpu.SemaphoreType.DMA,
    ],
)
def _gather_sc(idx_hbm, table_hbm, out_hbm, idx_v, rows_v, spbuf, sem_i, sem_s, sem_d):
    cid = lax.axis_index("c")
    sid = lax.axis_index("s")
    wid = sid * _NC + cid
    base = wid * _B_PER_W
    sp_base = sid * _HALF
    pltpu.async_copy(idx_hbm.at[pl.ds(base, _B_PER_W)], idx_v, sem_i).wait()

    def step(i, carry):
        t0 = i * 16
        vec_s = idx_v[pl.ds(t0, 16)]
        vec_d = idx_v[pl.ds(_HALF + t0, 16)]
        for j in range(16):
            rs = vec_s[j]
            pltpu.async_copy(
                table_hbm.at[pl.ds(rs, 1), :],
                rows_v.at[pl.ds(t0 + j, 1), :],
                sem_s,
            )
            rd = vec_d[j]
            pltpu.async_copy(
                table_hbm.at[pl.ds(rd, 1), :],
                spbuf.at[pl.ds(sp_base + t0 + j, 1), :],
                sem_d,
            )
        return carry

    lax.fori_loop(0, _HALF // 16, step, 0)
    pltpu.make_async_copy(
        table_hbm.at[pl.ds(0, _HALF), :], rows_v, sem_s
    ).wait()
    pltpu.make_async_copy(
        table_hbm.at[pl.ds(0, _HALF), :],
        spbuf.at[pl.ds(sp_base, _HALF), :],
        sem_d,
    ).wait()
    pltpu.sync_copy(rows_v, out_hbm.at[pl.ds(base, _HALF)])
    pltpu.sync_copy(
        spbuf.at[pl.ds(sp_base, _HALF)], out_hbm.at[pl.ds(base + _HALF, _HALF)]
    )


def kernel(instance_ids, embedding_instance):
    out = _gather_sc(instance_ids.astype(jnp.int32), embedding_instance)
    return out[None, ...]
